# baseline (speedup 1.0000x reference)
import functools

import jax
import jax.numpy as jnp
from jax import lax
from jax.experimental import pallas as pl
from jax.experimental.pallas import tpu as pltpu
from jax.experimental.pallas import tpu_sc as plsc

_NUM_FIELDS = 26
_CARD1 = 100001
_EMB = 64
_BATCH = 4096
_NW = 32
_ROWS_PER_W = _BATCH * _NUM_FIELDS // _NW
_CHUNK = 128
_NCHUNK = _ROWS_PER_W // _CHUNK
_LANES = 16


def _sc_embed(table_flat, x3):
    mesh = plsc.VectorSubcoreMesh(core_axis_name="c", subcore_axis_name="s")

    @functools.partial(
        pl.kernel,
        mesh=mesh,
        out_type=jax.ShapeDtypeStruct((_BATCH * _NUM_FIELDS, _EMB), jnp.float32),
        compiler_params=pltpu.CompilerParams(use_tc_tiling_on_sc=False),
        scratch_types=[
            pltpu.VMEM((_NCHUNK, _CHUNK), jnp.int32),
            pltpu.VMEM((_CHUNK, _EMB), jnp.float32),
            pltpu.VMEM((_CHUNK, _EMB), jnp.float32),
            pltpu.SemaphoreType.DMA,
            pltpu.SemaphoreType.DMA,
        ],
    )
    def k(table_hbm, x_hbm, out_hbm, idx_v, rows0, rows1, g0, g1):
        wid = lax.axis_index("s") * 2 + lax.axis_index("c")
        base = wid * _ROWS_PER_W

        pltpu.sync_copy(x_hbm.at[wid], idx_v)

        def add_offsets(j, carry):
            for i in range(_CHUNK // _LANES):
                sl = pl.ds(i * _LANES, _LANES)
                pos = lax.iota(jnp.int32, _LANES) + (j * _CHUNK + i * _LANES)
                off = lax.rem(pos, _NUM_FIELDS) * _CARD1
                idx_v[j, sl] = idx_v[j, sl] + off
            return carry

        lax.fori_loop(0, _NCHUNK, add_offsets, 0)

        bufs = (rows0, rows1)
        sems = (g0, g1)
        copies = [
            pltpu.async_copy(table_hbm.at[idx_v.at[j]], bufs[j], sems[j])
            for j in range(2)
        ]
        for j in range(_NCHUNK):
            b = j % 2
            copies[b].wait()
            pltpu.sync_copy(bufs[b], out_hbm.at[pl.ds(base + j * _CHUNK, _CHUNK)])
            if j + 2 < _NCHUNK:
                copies[b] = pltpu.async_copy(
                    table_hbm.at[idx_v.at[j + 2]], bufs[b], sems[b]
                )

    return k(table_flat, x3)


def kernel(x, tables):
    table_flat = tables.reshape(_NUM_FIELDS * _CARD1, _EMB)
    x3 = x.astype(jnp.int32).reshape(_NW, _NCHUNK, _CHUNK)
    out = _sc_embed(table_flat, x3)
    return out.reshape(_BATCH, _NUM_FIELDS * _EMB)
